# baseline (speedup 1.0000x reference)
import functools

import jax
import jax.numpy as jnp
from jax import lax
from jax.experimental import pallas as pl
from jax.experimental.pallas import tpu as pltpu
from jax.experimental.pallas import tpu_sc as plsc


def kernel(embed_user, embed_item):
    n_u, e = embed_user.shape
    n_i, _ = embed_item.shape
    info = plsc.get_sparse_core_info()
    nw = info.num_cores * info.num_subcores
    assert n_u == n_i, "workers assume equal table heights"
    n = n_u
    rows_main = -(-n // nw)
    rows_main += (-rows_main) % 8
    rows_last = n - (nw - 1) * rows_main
    assert rows_last > 0
    mesh = plsc.VectorSubcoreMesh(core_axis_name="c", subcore_axis_name="s")

    @functools.partial(
        pl.kernel,
        mesh=mesh,
        out_type=[
            jax.ShapeDtypeStruct((n_u, e), embed_user.dtype),
            jax.ShapeDtypeStruct((n_i, e), embed_item.dtype),
        ],
        scratch_types=[pltpu.SemaphoreType.DMA, pltpu.SemaphoreType.DMA],
    )
    def sc_copy(u_hbm, i_hbm, ou_hbm, oi_hbm, sem_u, sem_i):
        wid = lax.axis_index("s") * info.num_cores + lax.axis_index("c")
        base = pl.multiple_of(wid * rows_main, 8)

        @pl.when(wid < nw - 1)
        def _main():
            cp_u = pltpu.make_async_copy(
                u_hbm.at[pl.ds(base, rows_main)],
                ou_hbm.at[pl.ds(base, rows_main)], sem_u)
            cp_i = pltpu.make_async_copy(
                i_hbm.at[pl.ds(base, rows_main)],
                oi_hbm.at[pl.ds(base, rows_main)], sem_i)
            cp_u.start()
            cp_i.start()
            cp_u.wait()
            cp_i.wait()

        @pl.when(wid == nw - 1)
        def _tail():
            cp_u = pltpu.make_async_copy(
                u_hbm.at[pl.ds(base, rows_last)],
                ou_hbm.at[pl.ds(base, rows_last)], sem_u)
            cp_i = pltpu.make_async_copy(
                i_hbm.at[pl.ds(base, rows_last)],
                oi_hbm.at[pl.ds(base, rows_last)], sem_i)
            cp_u.start()
            cp_i.start()
            cp_u.wait()
            cp_i.wait()

    out_u, out_i = sc_copy(embed_user, embed_item)
    return (out_u, out_i)


# speedup vs baseline: 1.0052x; 1.0052x over previous
import jax
import jax.numpy as jnp
from jax.experimental import pallas as pl
from jax.experimental.pallas import tpu as pltpu

_SLICES = 8


def _slice_plan(n):
    chunk = -(-n // _SLICES)
    chunk += (-chunk) % 8
    plan = []
    off = 0
    while off < n:
        plan.append((off, min(chunk, n - off)))
        off += chunk
    return plan


def kernel(embed_user, embed_item):
    n_u, e = embed_user.shape
    n_i, _ = embed_item.shape
    plan_u = _slice_plan(n_u)
    plan_i = _slice_plan(n_i)
    n_sems = len(plan_u) + len(plan_i)

    def body(u_ref, i_ref, ou_ref, oi_ref, sems):
        copies = []
        for t, (off, sz) in enumerate(plan_u):
            copies.append(pltpu.make_async_copy(
                u_ref.at[pl.ds(off, sz)], ou_ref.at[pl.ds(off, sz)],
                sems.at[t]))
        for t, (off, sz) in enumerate(plan_i):
            copies.append(pltpu.make_async_copy(
                i_ref.at[pl.ds(off, sz)], oi_ref.at[pl.ds(off, sz)],
                sems.at[len(plan_u) + t]))
        for c in copies:
            c.start()
        for c in copies:
            c.wait()

    any_spec = pl.BlockSpec(memory_space=pltpu.MemorySpace.HBM)
    out_u, out_i = pl.pallas_call(
        body,
        in_specs=[any_spec, any_spec],
        out_specs=[any_spec, any_spec],
        out_shape=[
            jax.ShapeDtypeStruct((n_u, e), embed_user.dtype),
            jax.ShapeDtypeStruct((n_i, e), embed_item.dtype),
        ],
        scratch_shapes=[pltpu.SemaphoreType.DMA((n_sems,))],
    )(embed_user, embed_item)
    return (out_u, out_i)


# speedup vs baseline: 49.0222x; 48.7669x over previous
import jax
import jax.numpy as jnp
from jax.experimental import pallas as pl

_BLOCK_ROWS = 10000


def _copy_body(u_ref, i_ref, ou_ref, oi_ref):
    ou_ref[...] = u_ref[...]
    oi_ref[...] = i_ref[...]


def kernel(embed_user, embed_item):
    n_u, e = embed_user.shape
    n_i, _ = embed_item.shape
    assert n_u == n_i, "single-grid copy assumes equal table heights"
    grid = (n_u // _BLOCK_ROWS,)
    spec = pl.BlockSpec((_BLOCK_ROWS, e), lambda i: (i, 0))
    out_u, out_i = pl.pallas_call(
        _copy_body,
        grid=grid,
        in_specs=[spec, spec],
        out_specs=[spec, spec],
        out_shape=[
            jax.ShapeDtypeStruct((n_u, e), embed_user.dtype),
            jax.ShapeDtypeStruct((n_i, e), embed_item.dtype),
        ],
    )(embed_user, embed_item)
    return (out_u, out_i)


# speedup vs baseline: 49.8535x; 1.0170x over previous
import jax
import jax.numpy as jnp
from jax.experimental import pallas as pl
from jax.experimental.pallas import tpu as pltpu

_BLOCK_ROWS = 15000


def _copy_body(u_ref, i_ref, ou_ref, oi_ref):
    ou_ref[...] = u_ref[...]
    oi_ref[...] = i_ref[...]


def kernel(embed_user, embed_item):
    n_u, e = embed_user.shape
    n_i, _ = embed_item.shape
    assert n_u == n_i, "single-grid copy assumes equal table heights"
    grid = (-(-n_u // _BLOCK_ROWS),)
    spec = pl.BlockSpec((_BLOCK_ROWS, e), lambda i: (i, 0))
    out_u, out_i = pl.pallas_call(
        _copy_body,
        grid=grid,
        in_specs=[spec, spec],
        out_specs=[spec, spec],
        out_shape=[
            jax.ShapeDtypeStruct((n_u, e), embed_user.dtype),
            jax.ShapeDtypeStruct((n_i, e), embed_item.dtype),
        ],
        compiler_params=pltpu.CompilerParams(
            vmem_limit_bytes=128 * 1024 * 1024),
    )(embed_user, embed_item)
    return (out_u, out_i)
